# Initial kernel scaffold; baseline (speedup 1.0000x reference)
import jax
import jax.numpy as jnp
from jax.experimental import pallas as pl


def kernel(idx1, idx2, table):
    raise NotImplementedError("write your pallas kernel here")


# speedup vs baseline: 6.8343x; 6.8343x over previous
import functools

import jax
import jax.numpy as jnp
from jax import lax
from jax.experimental import pallas as pl
from jax.experimental.pallas import tpu as pltpu
from jax.experimental.pallas import tpu_sc as plsc

VOCAB = 100000
DIM = 64
B = 4096
L = 50
N = B * L

NUM_CORES = 2
NUM_SUBCORES = 16
LANES = 16
NW = NUM_CORES * NUM_SUBCORES
PAIRS_PER_WORKER = N // NW
CHUNK = 128
CHUNKS_PER_WORKER = PAIRS_PER_WORKER // CHUNK
GROUPS = CHUNK // LANES

_EPS2 = 1e-24


def _rsqrt(x):
    i = plsc.bitcast(x, jnp.int32)
    y = plsc.bitcast(jnp.int32(0x5F3759DF) - (i >> 1), jnp.float32)
    for _ in range(3):
        y = y * (1.5 - 0.5 * x * y * y)
    return y


def _body(idx1_hbm, idx2_hbm, table_hbm, out_hbm,
          idx1_v, idx2_v, r1_v, r2_v, out_v, dot_s, n1_s, n2_s, sem):
    wid = lax.axis_index("s") * NUM_CORES + lax.axis_index("c")
    base = wid * PAIRS_PER_WORKER

    def chunk_body(c, carry):
        off = base + c * CHUNK
        pltpu.sync_copy(idx1_hbm.at[pl.ds(off, CHUNK)], idx1_v)
        pltpu.sync_copy(idx2_hbm.at[pl.ds(off, CHUNK)], idx2_v)
        cp1 = pltpu.async_copy(table_hbm.at[idx1_v], r1_v, sem)
        cp2 = pltpu.async_copy(table_hbm.at[idx2_v], r2_v, sem)
        cp1.wait()
        cp2.wait()

        last_lane = lax.iota(jnp.int32, LANES) == (LANES - 1)

        def group_body(g, carry2):
            for u in range(LANES):
                p = g * LANES + u
                acc_d = None
                acc_1 = None
                acc_2 = None
                for k in range(DIM // LANES):
                    a = r1_v[p, pl.ds(k * LANES, LANES)]
                    b = r2_v[p, pl.ds(k * LANES, LANES)]
                    if acc_d is None:
                        acc_d, acc_1, acc_2 = a * b, a * a, b * b
                    else:
                        acc_d += a * b
                        acc_1 += a * a
                        acc_2 += b * b
                slot = jnp.full((LANES,), u, jnp.int32)
                plsc.store_scatter(dot_s, [slot], plsc.cumsum(acc_d),
                                   mask=last_lane)
                plsc.store_scatter(n1_s, [slot], plsc.cumsum(acc_1),
                                   mask=last_lane)
                plsc.store_scatter(n2_s, [slot], plsc.cumsum(acc_2),
                                   mask=last_lane)
            vd = dot_s[...]
            v1 = jnp.maximum(n1_s[...], _EPS2)
            v2 = jnp.maximum(n2_s[...], _EPS2)
            cos = vd * _rsqrt(v1) * _rsqrt(v2)
            out_v[pl.ds(c * CHUNK + g * LANES, LANES)] = cos
            return carry2

        return lax.fori_loop(0, GROUPS, group_body, carry)

    lax.fori_loop(0, CHUNKS_PER_WORKER, chunk_body, jnp.int32(0))

    pltpu.sync_copy(out_v, out_hbm.at[pl.ds(wid * PAIRS_PER_WORKER,
                                            PAIRS_PER_WORKER)])


@functools.partial(
    pl.kernel,
    out_type=jax.ShapeDtypeStruct((N,), jnp.float32),
    mesh=plsc.VectorSubcoreMesh(core_axis_name="c", subcore_axis_name="s"),
    compiler_params=pltpu.CompilerParams(
        needs_layout_passes=False, use_tc_tiling_on_sc=False
    ),
    scratch_types=[
        pltpu.VMEM((CHUNK,), jnp.int32),
        pltpu.VMEM((CHUNK,), jnp.int32),
        pltpu.VMEM((CHUNK, DIM), jnp.float32),
        pltpu.VMEM((CHUNK, DIM), jnp.float32),
        pltpu.VMEM((PAIRS_PER_WORKER,), jnp.float32),
        pltpu.VMEM((LANES,), jnp.float32),
        pltpu.VMEM((LANES,), jnp.float32),
        pltpu.VMEM((LANES,), jnp.float32),
        pltpu.SemaphoreType.DMA,
    ],
)
def _sc_cosine(idx1_hbm, idx2_hbm, table_hbm, out_hbm, *scratch):
    _body(idx1_hbm, idx2_hbm, table_hbm, out_hbm, *scratch)


def kernel(idx1, idx2, table):
    out = _sc_cosine(idx1.reshape(N), idx2.reshape(N), table)
    return out.reshape(B, L)
